# baseline (speedup 1.0000x reference)
import jax
import jax.numpy as jnp
from jax.experimental import pallas as pl
from jax.experimental.pallas import tpu as pltpu

_S = 4096
_D = 4096
_E = 64
_CAP = 64
_BS = 256
_GRID = _S // _BS


def _top1_kernel(x_ref, wg_ref, combine_ref, mask_ref, laux_ref,
                 counts_ref, me_ref, ce_ref):
    i = pl.program_id(0)

    @pl.when(i == 0)
    def _init():
        counts_ref[...] = jnp.zeros_like(counts_ref)
        me_ref[...] = jnp.zeros_like(me_ref)
        ce_ref[...] = jnp.zeros_like(ce_ref)

    logits = jax.lax.dot_general(
        x_ref[...], wg_ref[...], (((1,), (1,)), ((), ())),
        preferred_element_type=jnp.float32)
    gates = jax.nn.softmax(logits, axis=1)
    idx = jnp.argmax(gates, axis=1, keepdims=True)
    e_iota = jax.lax.broadcasted_iota(jnp.int32, (_BS, _E), 1)
    mask1 = (e_iota == idx).astype(jnp.float32)

    r_iota = jax.lax.broadcasted_iota(jnp.int32, (_BS, _BS), 0)
    c_iota = jax.lax.broadcasted_iota(jnp.int32, (_BS, _BS), 1)
    tri = (c_iota <= r_iota).astype(jnp.float32)
    csum = jax.lax.dot_general(
        tri, mask1, (((1,), (0,)), ((), ())),
        preferred_element_type=jnp.float32)

    counts = counts_ref[...]
    locations1 = csum - 1.0 + counts
    counts_ref[...] = counts + jnp.sum(mask1, axis=0, keepdims=True)

    me_ref[...] += jnp.sum(gates, axis=0, keepdims=True)
    ce_ref[...] += jnp.sum(mask1, axis=0, keepdims=True)

    loc = jnp.sum(locations1 * mask1, axis=1, keepdims=True)
    gate_s = jnp.sum(gates * mask1, axis=1, keepdims=True)
    val = jnp.where(loc < _CAP, gate_s, 0.0)
    pos = idx * _CAP + loc.astype(jnp.int32)

    k_iota = jax.lax.broadcasted_iota(jnp.int32, (_BS, _E * _CAP), 1)
    combine = jnp.where(k_iota == pos, val, 0.0)
    combine_ref[...] = combine
    mask_ref[...] = combine != 0.0

    @pl.when(i == _GRID - 1)
    def _fin():
        me = me_ref[...] * (1.0 / _S)
        ce = ce_ref[...] * (1.0 / _S)
        laux_ref[0, 0] = jnp.mean(me * ce) * (_E * _E)


def kernel(input_tensor, wg):
    combine2d, mask2d, laux = pl.pallas_call(
        _top1_kernel,
        grid=(_GRID,),
        in_specs=[
            pl.BlockSpec((_BS, _D), lambda i: (i, 0)),
            pl.BlockSpec((_E, _D), lambda i: (0, 0)),
        ],
        out_specs=[
            pl.BlockSpec((_BS, _E * _CAP), lambda i: (i, 0)),
            pl.BlockSpec((_BS, _E * _CAP), lambda i: (i, 0)),
            pl.BlockSpec((1, 1), lambda i: (0, 0), memory_space=pltpu.SMEM),
        ],
        out_shape=[
            jax.ShapeDtypeStruct((_S, _E * _CAP), jnp.float32),
            jax.ShapeDtypeStruct((_S, _E * _CAP), jnp.bool_),
            jax.ShapeDtypeStruct((1, 1), jnp.float32),
        ],
        scratch_shapes=[
            pltpu.VMEM((1, _E), jnp.float32),
            pltpu.VMEM((1, _E), jnp.float32),
            pltpu.VMEM((1, _E), jnp.float32),
        ],
    )(input_tensor, wg)
    combine = combine2d.reshape(_S, _E, _CAP)
    dispatch = mask2d.reshape(_S, _E, _CAP)
    return laux[0, 0], combine, dispatch
